# Initial kernel scaffold; baseline (speedup 1.0000x reference)
import functools

import jax
import jax.numpy as jnp
from jax import lax
from jax.experimental import pallas as pl
from jax.experimental.pallas import tpu as pltpu
from jax.experimental.pallas import tpu_sc as plsc

NC = 2
NS = 16
NW = NC * NS
L = 16

BN = 2000


@functools.lru_cache(maxsize=None)
def _deg_kernel(E, NPAD):
    EP = E // NW
    K = 125
    NCH = EP // K
    RPT = NPAD // NS

    mesh = plsc.VectorSubcoreMesh(core_axis_name="c", subcore_axis_name="s")

    @functools.partial(
        pl.kernel,
        out_type=jax.ShapeDtypeStruct((NC, NPAD), jnp.float32),
        mesh=mesh,
        scratch_types=[
            pltpu.VMEM((NCH, K), jnp.int32),
            pltpu.VMEM((128,), jnp.float32),
            pltpu.VMEM((RPT,), jnp.float32),
            pltpu.VMEM_SHARED((NPAD,), jnp.float32),
        ],
    )
    def deg_k(dst_hbm, out_hbm, dst_v, ones_v, z_v, acc_sh):
        c = lax.axis_index("c")
        s = lax.axis_index("s")
        wid = c * NS + s
        one16 = jnp.full((L,), 1.0, jnp.float32)
        zero16 = jnp.zeros((L,), jnp.float32)
        for k in range(128 // L):
            ones_v[pl.ds(k * L, L)] = one16
        for k in range(RPT // L):
            z_v[pl.ds(k * L, L)] = zero16
        pltpu.sync_copy(z_v, acc_sh.at[pl.ds(s * RPT, RPT)])
        pltpu.sync_copy(dst_hbm.at[wid], dst_v)
        plsc.subcore_barrier()

        @pl.loop(0, NCH)
        def _(j):
            pltpu.sync_copy(ones_v.at[pl.ds(0, K)], acc_sh.at[dst_v.at[j]],
                            add=True)

        plsc.subcore_barrier()
        pltpu.sync_copy(acc_sh.at[pl.ds(s * RPT, RPT)],
                        out_hbm.at[c, pl.ds(s * RPT, RPT)])

    return deg_k


@functools.lru_cache(maxsize=None)
def _scatter_kernel(E, NPAD, D):
    EP = E // NW
    K = 125
    NCH = EP // K
    RPT = NPAD // NS
    ZR = 32

    mesh = plsc.VectorSubcoreMesh(core_axis_name="c", subcore_axis_name="s")

    @functools.partial(
        pl.kernel,
        out_type=jax.ShapeDtypeStruct((NC, NPAD, D), jnp.float32),
        mesh=mesh,
        scratch_types=[
            pltpu.VMEM((NCH, K), jnp.int32),
            pltpu.VMEM((NCH, K), jnp.int32),
            pltpu.VMEM((K, D), jnp.float32),
            pltpu.VMEM((ZR, D), jnp.float32),
            pltpu.VMEM_SHARED((NPAD, D), jnp.float32),
        ],
    )
    def scat_k(y_hbm, src_hbm, dst_hbm, out_hbm,
               src_v, dst_v, rows_v, z_v, acc_sh):
        c = lax.axis_index("c")
        s = lax.axis_index("s")
        wid = c * NS + s
        zero16 = jnp.zeros((L,), jnp.float32)

        @pl.loop(0, ZR)
        def _(r):
            for k in range(D // L):
                z_v[r, pl.ds(k * L, L)] = zero16

        @pl.loop(0, RPT // ZR)
        def _(i):
            pltpu.sync_copy(z_v, acc_sh.at[pl.ds(s * RPT + i * ZR, ZR)])

        pltpu.sync_copy(src_hbm.at[wid], src_v)
        pltpu.sync_copy(dst_hbm.at[wid], dst_v)
        plsc.subcore_barrier()

        @pl.loop(0, NCH)
        def _(j):
            pltpu.sync_copy(y_hbm.at[src_v.at[j]], rows_v)
            pltpu.sync_copy(rows_v, acc_sh.at[dst_v.at[j]], add=True)

        plsc.subcore_barrier()
        pltpu.sync_copy(acc_sh.at[pl.ds(s * RPT, RPT)],
                        out_hbm.at[c, pl.ds(s * RPT, RPT)])

    return scat_k


def _tc_matmul(x, W):
    N_, DI = x.shape
    DO = W.shape[1]

    def body(x_ref, w_ref, o_ref):
        o_ref[...] = jnp.dot(x_ref[...], w_ref[...],
                             preferred_element_type=jnp.float32)

    return pl.pallas_call(
        body,
        grid=(N_ // BN,),
        in_specs=[
            pl.BlockSpec((BN, DI), lambda i: (i, 0)),
            pl.BlockSpec((DI, DO), lambda i: (0, 0)),
        ],
        out_specs=pl.BlockSpec((BN, DO), lambda i: (i, 0)),
        out_shape=jax.ShapeDtypeStruct((N_, DO), jnp.float32),
    )(x, W)


def _tc_dinv_scale(degp_t, xw):
    N_, D = xw.shape

    def body(dp_ref, xw_ref, y_ref, dinv_ref):
        deg = jnp.sum(dp_ref[...], axis=1, keepdims=True) + 1.0
        dinv = lax.rsqrt(jnp.maximum(deg, 1e-12))
        dinv_ref[...] = dinv
        y_ref[...] = xw_ref[...] * dinv

    return pl.pallas_call(
        body,
        grid=(N_ // BN,),
        in_specs=[
            pl.BlockSpec((BN, NC), lambda i: (i, 0)),
            pl.BlockSpec((BN, D), lambda i: (i, 0)),
        ],
        out_specs=[
            pl.BlockSpec((BN, D), lambda i: (i, 0)),
            pl.BlockSpec((BN, 1), lambda i: (i, 0)),
        ],
        out_shape=[
            jax.ShapeDtypeStruct((N_, D), jnp.float32),
            jax.ShapeDtypeStruct((N_, 1), jnp.float32),
        ],
    )(degp_t, xw)


def _tc_combine_matmul(parts, y, dinv, b, W, scale_out):
    N_, D = y.shape
    DO = W.shape[1]

    def body(p_ref, y_ref, dinv_ref, b_ref, w_ref, o_ref):
        S = p_ref[0] + p_ref[1] + y_ref[...]
        h = jnp.maximum(S * dinv_ref[...] + b_ref[...], 0.0)
        o = jnp.dot(h, w_ref[...], preferred_element_type=jnp.float32)
        if scale_out:
            o = o * dinv_ref[...]
        o_ref[...] = o

    return pl.pallas_call(
        body,
        grid=(N_ // BN,),
        in_specs=[
            pl.BlockSpec((NC, BN, D), lambda i: (0, i, 0)),
            pl.BlockSpec((BN, D), lambda i: (i, 0)),
            pl.BlockSpec((BN, 1), lambda i: (i, 0)),
            pl.BlockSpec((1, D), lambda i: (0, 0)),
            pl.BlockSpec((D, DO), lambda i: (0, 0)),
        ],
        out_specs=pl.BlockSpec((BN, DO), lambda i: (i, 0)),
        out_shape=jax.ShapeDtypeStruct((N_, DO), jnp.float32),
    )(parts, y, dinv, b, W)


def _tc_final(parts, y, dinv, b, W, b_out):
    N_, D = y.shape
    DO = W.shape[1]

    def body(p_ref, y_ref, dinv_ref, b_ref, w_ref, bo_ref, o_ref):
        S = p_ref[0] + p_ref[1] + y_ref[...]
        h = jnp.maximum(S * dinv_ref[...] + b_ref[...], 0.0)
        o_ref[...] = jnp.dot(h, w_ref[...],
                             preferred_element_type=jnp.float32) + bo_ref[...]

    return pl.pallas_call(
        body,
        grid=(N_ // BN,),
        in_specs=[
            pl.BlockSpec((NC, BN, D), lambda i: (0, i, 0)),
            pl.BlockSpec((BN, D), lambda i: (i, 0)),
            pl.BlockSpec((BN, 1), lambda i: (i, 0)),
            pl.BlockSpec((1, D), lambda i: (0, 0)),
            pl.BlockSpec((D, DO), lambda i: (0, 0)),
            pl.BlockSpec((1, DO), lambda i: (0, 0)),
        ],
        out_specs=pl.BlockSpec((BN, DO), lambda i: (i, 0)),
        out_shape=jax.ShapeDtypeStruct((N_, DO), jnp.float32),
    )(parts, y, dinv, b, W, b_out)


def kernel(x, edge_index, W1, b1, W2, b2, W_out, b_out):
    N_, D_in = x.shape
    E = edge_index.shape[1]
    assert E % NW == 0 and (E // NW) % 125 == 0
    NPAD = ((N_ + NS * L - 1) // (NS * L)) * (NS * L)

    EP = E // NW
    K = 125
    src3 = edge_index[0].reshape(NW, EP // K, K)
    dst3 = edge_index[1].reshape(NW, EP // K, K)

    degp = _deg_kernel(E, NPAD)(dst3)
    xw1 = _tc_matmul(x, W1)
    degp_t = degp.T[:N_]

    y1, dinv = _tc_dinv_scale(degp_t, xw1)

    parts1 = _scatter_kernel(E, NPAD, y1.shape[1])(y1, src3, dst3)
    parts1 = parts1[:, :N_]

    y2 = _tc_combine_matmul(parts1, y1, dinv, b1.reshape(1, -1), W2, True)

    parts2 = _scatter_kernel(E, NPAD, y2.shape[1])(y2, src3, dst3)
    parts2 = parts2[:, :N_]

    return _tc_final(parts2, y2, dinv, b2.reshape(1, -1), W_out,
                     b_out.reshape(1, -1))


# speedup vs baseline: 22.3539x; 22.3539x over previous
import functools

import jax
import jax.numpy as jnp
from jax import lax
from jax.experimental import pallas as pl
from jax.experimental.pallas import tpu as pltpu
from jax.experimental.pallas import tpu_sc as plsc

NC = 2
NS = 16
NW = NC * NS
L = 16

BN = 2000


@functools.lru_cache(maxsize=None)
def _deg_kernel(E, NPAD):
    EP = E // NW
    K = 125
    NCH = EP // K
    RPT = NPAD // NS

    mesh = plsc.VectorSubcoreMesh(core_axis_name="c", subcore_axis_name="s")

    @functools.partial(
        pl.kernel,
        out_type=jax.ShapeDtypeStruct((NC, NPAD), jnp.float32),
        mesh=mesh,
        scratch_types=[
            pltpu.VMEM((NCH, K), jnp.int32),
            pltpu.VMEM((128,), jnp.float32),
            pltpu.VMEM((RPT,), jnp.float32),
            pltpu.VMEM_SHARED((NPAD,), jnp.float32),
        ],
    )
    def deg_k(dst_hbm, out_hbm, dst_v, ones_v, z_v, acc_sh):
        c = lax.axis_index("c")
        s = lax.axis_index("s")
        wid = c * NS + s
        one16 = jnp.full((L,), 1.0, jnp.float32)
        zero16 = jnp.zeros((L,), jnp.float32)
        for k in range(128 // L):
            ones_v[pl.ds(k * L, L)] = one16
        for k in range(RPT // L):
            z_v[pl.ds(k * L, L)] = zero16
        pltpu.sync_copy(z_v, acc_sh.at[pl.ds(s * RPT, RPT)])
        pltpu.sync_copy(dst_hbm.at[wid], dst_v)
        plsc.subcore_barrier()

        @pl.loop(0, NCH)
        def _(j):
            pltpu.sync_copy(ones_v.at[pl.ds(0, K)], acc_sh.at[dst_v.at[j]],
                            add=True)

        plsc.subcore_barrier()
        pltpu.sync_copy(acc_sh.at[pl.ds(s * RPT, RPT)],
                        out_hbm.at[c, pl.ds(s * RPT, RPT)])

    return deg_k


@functools.lru_cache(maxsize=None)
def _scatter_kernel(E, NPAD, D):
    EP = E // NW
    K = 125
    NCH = EP // K
    RPT = NPAD // NS
    ZR = 32

    mesh = plsc.VectorSubcoreMesh(core_axis_name="c", subcore_axis_name="s")

    @functools.partial(
        pl.kernel,
        out_type=jax.ShapeDtypeStruct((NC, NPAD, D), jnp.float32),
        mesh=mesh,
        scratch_types=[
            pltpu.VMEM((NCH, K), jnp.int32),
            pltpu.VMEM((NCH, K), jnp.int32),
            pltpu.VMEM((K, D), jnp.float32),
            pltpu.VMEM((ZR, D), jnp.float32),
            pltpu.VMEM_SHARED((NPAD, D), jnp.float32),
        ],
    )
    def scat_k(y_hbm, src_hbm, dst_hbm, out_hbm,
               src_v, dst_v, rows_v, z_v, acc_sh):
        c = lax.axis_index("c")
        s = lax.axis_index("s")
        wid = c * NS + s
        zero16 = jnp.zeros((L,), jnp.float32)

        @pl.loop(0, ZR)
        def _(r):
            for k in range(D // L):
                z_v[r, pl.ds(k * L, L)] = zero16

        @pl.loop(0, RPT // ZR)
        def _(i):
            pltpu.sync_copy(z_v, acc_sh.at[pl.ds(s * RPT + i * ZR, ZR)])

        pltpu.sync_copy(src_hbm.at[wid], src_v)
        pltpu.sync_copy(dst_hbm.at[wid], dst_v)
        plsc.subcore_barrier()

        @pl.loop(0, NCH)
        def _(j):
            pltpu.sync_copy(y_hbm.at[src_v.at[j]], rows_v)
            pltpu.sync_copy(rows_v, acc_sh.at[dst_v.at[j]], add=True)

        plsc.subcore_barrier()
        pltpu.sync_copy(acc_sh.at[pl.ds(s * RPT, RPT)],
                        out_hbm.at[c, pl.ds(s * RPT, RPT)])

    return scat_k


def _tc_matmul(x, W):
    N_, DI = x.shape
    DO = W.shape[1]

    def body(x_ref, w_ref, o_ref):
        o_ref[...] = jnp.dot(x_ref[...], w_ref[...],
                             preferred_element_type=jnp.float32)

    return pl.pallas_call(
        body,
        grid=(N_ // BN,),
        in_specs=[
            pl.BlockSpec((BN, DI), lambda i: (i, 0)),
            pl.BlockSpec((DI, DO), lambda i: (0, 0)),
        ],
        out_specs=pl.BlockSpec((BN, DO), lambda i: (i, 0)),
        out_shape=jax.ShapeDtypeStruct((N_, DO), jnp.float32),
    )(x, W)


def _tc_dinv_scale(degp_t, xw):
    N_, D = xw.shape

    def body(dp_ref, xw_ref, y_ref, dinv_ref):
        deg = jnp.sum(dp_ref[...], axis=1, keepdims=True) + 1.0
        dinv = lax.rsqrt(jnp.maximum(deg, 1e-12))
        dinv_ref[...] = dinv
        y_ref[...] = xw_ref[...] * dinv

    return pl.pallas_call(
        body,
        grid=(N_ // BN,),
        in_specs=[
            pl.BlockSpec((BN, NC), lambda i: (i, 0)),
            pl.BlockSpec((BN, D), lambda i: (i, 0)),
        ],
        out_specs=[
            pl.BlockSpec((BN, D), lambda i: (i, 0)),
            pl.BlockSpec((BN, 1), lambda i: (i, 0)),
        ],
        out_shape=[
            jax.ShapeDtypeStruct((N_, D), jnp.float32),
            jax.ShapeDtypeStruct((N_, 1), jnp.float32),
        ],
    )(degp_t, xw)


def _tc_combine_matmul(parts, y, dinv, b, W, scale_out):
    N_, D = y.shape
    DO = W.shape[1]

    def body(p_ref, y_ref, dinv_ref, b_ref, w_ref, o_ref):
        S = p_ref[0] + p_ref[1] + y_ref[...]
        h = jnp.maximum(S * dinv_ref[...] + b_ref[...], 0.0)
        o = jnp.dot(h, w_ref[...], preferred_element_type=jnp.float32)
        if scale_out:
            o = o * dinv_ref[...]
        o_ref[...] = o

    return pl.pallas_call(
        body,
        grid=(N_ // BN,),
        in_specs=[
            pl.BlockSpec((NC, BN, D), lambda i: (0, i, 0)),
            pl.BlockSpec((BN, D), lambda i: (i, 0)),
            pl.BlockSpec((BN, 1), lambda i: (i, 0)),
            pl.BlockSpec((1, D), lambda i: (0, 0)),
            pl.BlockSpec((D, DO), lambda i: (0, 0)),
        ],
        out_specs=pl.BlockSpec((BN, DO), lambda i: (i, 0)),
        out_shape=jax.ShapeDtypeStruct((N_, DO), jnp.float32),
    )(parts, y, dinv, b, W)


def _tc_final(parts, y, dinv, b, W, b_out):
    N_, D = y.shape
    DO = W.shape[1]

    def body(p_ref, y_ref, dinv_ref, b_ref, w_ref, bo_ref, o_ref):
        S = p_ref[0] + p_ref[1] + y_ref[...]
        h = jnp.maximum(S * dinv_ref[...] + b_ref[...], 0.0)
        o_ref[...] = jnp.dot(h, w_ref[...],
                             preferred_element_type=jnp.float32) + bo_ref[...]

    return pl.pallas_call(
        body,
        grid=(N_ // BN,),
        in_specs=[
            pl.BlockSpec((NC, BN, D), lambda i: (0, i, 0)),
            pl.BlockSpec((BN, D), lambda i: (i, 0)),
            pl.BlockSpec((BN, 1), lambda i: (i, 0)),
            pl.BlockSpec((1, D), lambda i: (0, 0)),
            pl.BlockSpec((D, DO), lambda i: (0, 0)),
            pl.BlockSpec((1, DO), lambda i: (0, 0)),
        ],
        out_specs=pl.BlockSpec((BN, DO), lambda i: (i, 0)),
        out_shape=jax.ShapeDtypeStruct((N_, DO), jnp.float32),
    )(parts, y, dinv, b, W, b_out)


def kernel(x, edge_index, W1, b1, W2, b2, W_out, b_out):
    N_, D_in = x.shape
    E = edge_index.shape[1]
    assert E % NW == 0 and (E // NW) % 125 == 0
    NPAD = ((N_ + NS * L - 1) // (NS * L)) * (NS * L)

    EP = E // NW
    K = 125
    src3 = edge_index[0].reshape(NW, EP // K, K)
    dst3 = edge_index[1].reshape(NW, EP // K, K)

    D2 = W2.shape[1]
    DP = ((D2 + 127) // 128) * 128
    if DP != D2:
        W2 = jnp.pad(W2, ((0, 0), (0, DP - D2)))
        b2 = jnp.pad(b2, (0, DP - D2))
        W_out = jnp.pad(W_out, ((0, DP - D2), (0, 0)))

    degp = _deg_kernel(E, NPAD)(dst3)
    xw1 = _tc_matmul(x, W1)
    degp_t = degp.T[:N_]

    y1, dinv = _tc_dinv_scale(degp_t, xw1)

    parts1 = _scatter_kernel(E, NPAD, y1.shape[1])(y1, src3, dst3)
    parts1 = parts1[:, :N_]

    y2 = _tc_combine_matmul(parts1, y1, dinv, b1.reshape(1, -1), W2, True)

    parts2 = _scatter_kernel(E, NPAD, y2.shape[1])(y2, src3, dst3)
    parts2 = parts2[:, :N_]

    return _tc_final(parts2, y2, dinv, b2.reshape(1, -1), W_out,
                     b_out.reshape(1, -1))
